# Initial kernel scaffold; baseline (speedup 1.0000x reference)
import jax
import jax.numpy as jnp
from jax.experimental import pallas as pl


def kernel(p_matrix, exer_emb, exer_lam, concept_emb, Q_matrix, resp_emb, Wq, bq, Wk, bk, Wv, bv, er_W, er_b, map_W, map_b):
    raise NotImplementedError("write your pallas kernel here")


# speedup vs baseline: 131.2988x; 131.2988x over previous
import jax
import jax.numpy as jnp
from jax.experimental import pallas as pl

B, N_EX, N_CON, D, H, OUT = 8, 2048, 128, 128, 4, 256
DH = D // H


def _enc_kernel(p_ref, exer_ref, lam_ref, concept_ref, q_ref, resp_ref,
                wq_ref, bq_ref, wk_ref, bk_ref, wv_ref, bv_ref,
                mapw_ref, mapb_ref, out_ref):
    f32 = jnp.float32
    Qm = q_ref[...]
    csum = jnp.dot(Qm, concept_ref[...], preferred_element_type=f32)
    ccnt = jnp.sum(Qm, axis=1, keepdims=True)
    rasch = exer_ref[...] + lam_ref[...] * (csum / ccnt)

    mk = jnp.dot(rasch, wk_ref[...], preferred_element_type=f32) + bk_ref[...]
    resp = resp_ref[...]
    mq = jnp.dot(resp, wq_ref[...], preferred_element_type=f32) + bq_ref[...]
    mv = jnp.dot(resp, wv_ref[...], preferred_element_type=f32) + bv_ref[...]

    rows = jax.lax.broadcasted_iota(jnp.int32, (D, H), 0)
    cols = jax.lax.broadcasted_iota(jnp.int32, (D, H), 1)
    sel = (rows // DH == cols).astype(f32)
    rowsT = jax.lax.broadcasted_iota(jnp.int32, (H, D), 0)
    colsT = jax.lax.broadcasted_iota(jnp.int32, (H, D), 1)
    selT = (colsT // DH == rowsT).astype(f32)

    scale = 1.0 / (DH ** 0.5)
    s1 = jnp.dot(mk * mq[1:2, :], sel, preferred_element_type=f32) * scale
    s2 = jnp.dot(mk * mq[2:3, :], sel, preferred_element_type=f32) * scale
    w1 = jnp.exp(s1 - jnp.max(s1, axis=0, keepdims=True))
    w2 = jnp.exp(s2 - jnp.max(s2, axis=0, keepdims=True))

    p = p_ref[...]
    ind1 = (p == 1).astype(f32)
    ind2 = (p == 2).astype(f32)
    e11 = jnp.dot(ind1, w1, preferred_element_type=f32)
    e12 = jnp.dot(ind2, w1, preferred_element_type=f32)
    e21 = jnp.dot(ind1, w2, preferred_element_type=f32)
    e22 = jnp.dot(ind2, w2, preferred_element_type=f32)

    d1 = e11 + e12
    d2 = e21 + e22
    sd1 = jnp.where(d1 > 0.0, d1, 1.0)
    sd2 = jnp.where(d2 > 0.0, d2, 1.0)

    v1 = mv[1:2, :]
    v2 = mv[2:3, :]
    theta1 = (jnp.dot(e11 / sd1, selT, preferred_element_type=f32) * v1
              + jnp.dot(e12 / sd1, selT, preferred_element_type=f32) * v2)
    theta2 = (jnp.dot(e21 / sd2, selT, preferred_element_type=f32) * v1
              + jnp.dot(e22 / sd2, selT, preferred_element_type=f32) * v2)

    n1 = jnp.sum(ind1, axis=1, keepdims=True)
    n2 = jnp.sum(ind2, axis=1, keepdims=True)
    avg = (n1 * theta1 + n2 * theta2) / jnp.maximum(n1 + n2, 1.0)
    logits = jnp.dot(avg, mapw_ref[...], preferred_element_type=f32) + mapb_ref[...]
    out_ref[...] = jax.nn.sigmoid(logits)


def kernel(p_matrix, exer_emb, exer_lam, concept_emb, Q_matrix, resp_emb,
           Wq, bq, Wk, bk, Wv, bv, er_W, er_b, map_W, map_b):
    del er_W, er_b
    args = (p_matrix.astype(jnp.int32), exer_emb, exer_lam, concept_emb,
            Q_matrix, resp_emb,
            Wq, bq.reshape(1, D), Wk, bk.reshape(1, D), Wv, bv.reshape(1, D),
            map_W, map_b.reshape(1, OUT))
    return pl.pallas_call(
        _enc_kernel,
        out_shape=jax.ShapeDtypeStruct((B, OUT), jnp.float32),
    )(*args)
